# Initial kernel scaffold; baseline (speedup 1.0000x reference)
import jax
import jax.numpy as jnp
from jax.experimental import pallas as pl


def kernel(pos_ids, pe):
    raise NotImplementedError("write your pallas kernel here")


# speedup vs baseline: 2.3823x; 2.3823x over previous
import functools

import jax
import jax.numpy as jnp
from jax import lax
from jax.experimental import pallas as pl
from jax.experimental.pallas import tpu as pltpu
from jax.experimental.pallas import tpu_sc as plsc

D_MODEL = 1024
N_ROWS = 4 * 8192

_NC = 2
_NS = 16
_NW = _NC * _NS

_B_PER_W = N_ROWS // _NW
_CHUNK = 32
_NCHUNK = _B_PER_W // _CHUNK
_NPAIR = _NCHUNK // 2


def _sc_gather(idx_hbm, table_hbm, out_hbm, idx_v, buf0, buf1, sem0, sem1):
    wid = lax.axis_index("s") * _NC + lax.axis_index("c")
    base = wid * _B_PER_W

    pltpu.sync_copy(idx_hbm.at[pl.ds(base, _B_PER_W)], idx_v)

    bufs = (buf0, buf1)
    sems = (sem0, sem1)

    def start(c, b):
        pltpu.async_copy(
            table_hbm.at[idx_v.at[pl.ds(c * _CHUNK, _CHUNK)]], bufs[b], sems[b]
        )

    def finish(c, b):
        pltpu.make_async_copy(
            table_hbm.at[idx_v.at[pl.ds(c * _CHUNK, _CHUNK)]], bufs[b], sems[b]
        ).wait()
        pltpu.sync_copy(bufs[b], out_hbm.at[pl.ds(base + c * _CHUNK, _CHUNK)])

    start(0, 0)
    start(1, 1)

    def pair_body(p, carry):
        for b in range(2):
            c = 2 * p + b
            finish(c, b)
            start(c + 2, b)
        return carry

    lax.fori_loop(0, _NPAIR - 1, pair_body, 0)

    for b in range(2):
        finish(2 * (_NPAIR - 1) + b, b)


@functools.partial(jax.jit, static_argnames=())
def kernel(pos_ids, pe):
    batch, seq_len = pos_ids.shape
    idx = pos_ids.reshape(-1).astype(jnp.int32)

    mesh = plsc.VectorSubcoreMesh(core_axis_name="c", subcore_axis_name="s")
    out = pl.kernel(
        _sc_gather,
        out_type=jax.ShapeDtypeStruct((N_ROWS, D_MODEL), jnp.float32),
        mesh=mesh,
        scratch_types=[
            pltpu.VMEM((_B_PER_W,), jnp.int32),
            pltpu.VMEM((_CHUNK, D_MODEL), jnp.float32),
            pltpu.VMEM((_CHUNK, D_MODEL), jnp.float32),
            pltpu.SemaphoreType.DMA,
            pltpu.SemaphoreType.DMA,
        ],
    )(idx, pe)

    return out.reshape(batch, seq_len, D_MODEL)


# speedup vs baseline: 2.3886x; 1.0027x over previous
import functools

import jax
import jax.numpy as jnp
from jax import lax
from jax.experimental import pallas as pl
from jax.experimental.pallas import tpu as pltpu
from jax.experimental.pallas import tpu_sc as plsc

D_MODEL = 1024
N_ROWS = 4 * 8192

_NC = 2
_NS = 16
_NW = _NC * _NS

_B_PER_W = N_ROWS // _NW
_CHUNK = 32
_NCHUNK = _B_PER_W // _CHUNK
_NPAIR = _NCHUNK // 2


def _sc_gather(
    idx_hbm, table_hbm, out_hbm, idx_v, buf0, buf1, gsem0, gsem1, ssem0, ssem1
):
    wid = lax.axis_index("s") * _NC + lax.axis_index("c")
    base = wid * _B_PER_W

    pltpu.sync_copy(idx_hbm.at[pl.ds(base, _B_PER_W)], idx_v)

    bufs = (buf0, buf1)
    gsems = (gsem0, gsem1)
    ssems = (ssem0, ssem1)

    def start_gather(c, b):
        pltpu.async_copy(
            table_hbm.at[idx_v.at[pl.ds(c * _CHUNK, _CHUNK)]], bufs[b], gsems[b]
        )

    def wait_gather(c, b):
        pltpu.make_async_copy(
            table_hbm.at[idx_v.at[pl.ds(c * _CHUNK, _CHUNK)]], bufs[b], gsems[b]
        ).wait()

    def start_store(c, b):
        pltpu.async_copy(
            bufs[b], out_hbm.at[pl.ds(base + c * _CHUNK, _CHUNK)], ssems[b]
        )

    def wait_store(c, b):
        pltpu.make_async_copy(
            bufs[b], out_hbm.at[pl.ds(base + c * _CHUNK, _CHUNK)], ssems[b]
        ).wait()

    start_gather(0, 0)

    start_gather(1, 1)
    wait_gather(0, 0)
    start_store(0, 0)

    def pair_body(p, carry):
        for b in (1, 0):
            c = 2 * p + (1 if b == 1 else 2)
            other = 1 - b
            wait_store(c - 1, other)
            start_gather(c + 1, other)
            wait_gather(c, b)
            start_store(c, b)
        return carry

    lax.fori_loop(0, (_NCHUNK - 2) // 2, pair_body, 0)

    c_last = _NCHUNK - 1
    wait_store(c_last - 1, 0)
    wait_gather(c_last, 1)
    start_store(c_last, 1)
    wait_store(c_last, 1)


@functools.partial(jax.jit, static_argnames=())
def kernel(pos_ids, pe):
    batch, seq_len = pos_ids.shape
    idx = pos_ids.reshape(-1).astype(jnp.int32)

    mesh = plsc.VectorSubcoreMesh(core_axis_name="c", subcore_axis_name="s")
    out = pl.kernel(
        _sc_gather,
        out_type=jax.ShapeDtypeStruct((N_ROWS, D_MODEL), jnp.float32),
        mesh=mesh,
        scratch_types=[
            pltpu.VMEM((_B_PER_W,), jnp.int32),
            pltpu.VMEM((_CHUNK, D_MODEL), jnp.float32),
            pltpu.VMEM((_CHUNK, D_MODEL), jnp.float32),
            pltpu.SemaphoreType.DMA,
            pltpu.SemaphoreType.DMA,
            pltpu.SemaphoreType.DMA,
            pltpu.SemaphoreType.DMA,
        ],
    )(idx, pe)

    return out.reshape(batch, seq_len, D_MODEL)


# speedup vs baseline: 2.3904x; 1.0007x over previous
import functools

import jax
import jax.numpy as jnp
from jax import lax
from jax.experimental import pallas as pl
from jax.experimental.pallas import tpu as pltpu
from jax.experimental.pallas import tpu_sc as plsc

D_MODEL = 1024
N_ROWS = 4 * 8192

_NC = 2
_NS = 16
_NW = _NC * _NS

_B_PER_W = N_ROWS // _NW
_CHUNK = 16
_NB = 4
_NCHUNK = _B_PER_W // _CHUNK
_LAG = 2


def _sc_gather(idx_hbm, table_hbm, out_hbm, idx_v, bufs, gsems, ssems):
    wid = lax.axis_index("s") * _NC + lax.axis_index("c")
    base = wid * _B_PER_W

    pltpu.sync_copy(idx_hbm.at[pl.ds(base, _B_PER_W)], idx_v)

    def start_gather(c, b):
        pltpu.async_copy(
            table_hbm.at[idx_v.at[pl.ds(c * _CHUNK, _CHUNK)]], bufs[b], gsems[b]
        )

    def wait_gather(c, b):
        pltpu.make_async_copy(
            table_hbm.at[idx_v.at[pl.ds(c * _CHUNK, _CHUNK)]], bufs[b], gsems[b]
        ).wait()

    def start_store(c, b):
        pltpu.async_copy(
            bufs[b], out_hbm.at[pl.ds(base + c * _CHUNK, _CHUNK)], ssems[b]
        )

    def wait_store(c, b):
        pltpu.make_async_copy(
            bufs[b], out_hbm.at[pl.ds(base + c * _CHUNK, _CHUNK)], ssems[b]
        ).wait()

    for c in range(_NB):
        start_gather(c, c % _NB)
        if c >= _LAG:
            wait_gather(c - _LAG, (c - _LAG) % _NB)
            start_store(c - _LAG, (c - _LAG) % _NB)

    def group_body(p, carry):
        for b in range(_NB):
            c = p * _NB + b
            bl = (b - _LAG) % _NB
            wait_store(c - _NB, b)
            start_gather(c, b)
            wait_gather(c - _LAG, bl)
            start_store(c - _LAG, bl)
        return carry

    lax.fori_loop(1, _NCHUNK // _NB, group_body, 0)

    for c in range(_NCHUNK, _NCHUNK + _LAG):
        wait_gather(c - _LAG, (c - _LAG) % _NB)
        start_store(c - _LAG, (c - _LAG) % _NB)
    for c in range(_NCHUNK - _NB, _NCHUNK):
        wait_store(c, c % _NB)


@functools.partial(jax.jit, static_argnames=())
def kernel(pos_ids, pe):
    batch, seq_len = pos_ids.shape
    idx = pos_ids.reshape(-1).astype(jnp.int32)

    mesh = plsc.VectorSubcoreMesh(core_axis_name="c", subcore_axis_name="s")
    out = pl.kernel(
        _sc_gather,
        out_type=jax.ShapeDtypeStruct((N_ROWS, D_MODEL), jnp.float32),
        mesh=mesh,
        scratch_types=[
            pltpu.VMEM((_B_PER_W,), jnp.int32),
            [pltpu.VMEM((_CHUNK, D_MODEL), jnp.float32) for _ in range(_NB)],
            [pltpu.SemaphoreType.DMA for _ in range(_NB)],
            [pltpu.SemaphoreType.DMA for _ in range(_NB)],
        ],
    )(idx, pe)

    return out.reshape(batch, seq_len, D_MODEL)
